# baseline (speedup 1.0000x reference)
import functools

import jax
import jax.numpy as jnp
from jax import lax
from jax.experimental import pallas as pl
from jax.experimental.pallas import tpu as pltpu
from jax.experimental.pallas import tpu_sc as plsc

BATCH = 16384
EMBED_DIM = 32
ROW = 4 * EMBED_DIM
NUM_TABLE_ROWS = 100000
NUM_CORES = 2
NUM_SUBCORES = 16
NUM_WORKERS = NUM_CORES * NUM_SUBCORES
ROWS_PER_WORKER = BATCH // NUM_WORKERS


def _pack_body(ug_r, um_r, ig_r, im_r, out_r):
    dn = (((0,), (0,)), ((), ()))

    def tr(x_r, t, dt):
        eye = jnp.eye(EMBED_DIM, ROW, EMBED_DIM * t, dtype=dt)
        return jax.lax.dot_general(x_r[...].astype(dt), eye,
                                   dimension_numbers=dn,
                                   preferred_element_type=jnp.float32)

    f32, bf16 = jnp.float32, jnp.bfloat16
    out_r[...] = ((tr(ug_r, 0, f32) + tr(um_r, 1, bf16))
                  + (tr(ig_r, 2, f32) + tr(im_r, 3, bf16)))


def _tc_pack(ug, um, ig, im, block_rows=8192):
    n_blocks = (NUM_TABLE_ROWS + block_rows - 1) // block_rows
    in_spec = pl.BlockSpec((EMBED_DIM, block_rows), lambda i: (0, i))
    return pl.pallas_call(
        _pack_body,
        grid=(n_blocks,),
        in_specs=[in_spec, in_spec, in_spec, in_spec],
        out_specs=pl.BlockSpec((block_rows, ROW), lambda i: (i, 0)),
        out_shape=jax.ShapeDtypeStruct((NUM_TABLE_ROWS, ROW), jnp.float32),
        compiler_params=pltpu.CompilerParams(
            dimension_semantics=("parallel",)),
    )(ug.T, um.T, ig.T, im.T)


def _sc_gather2(uidx, iidx, table):
    mesh = plsc.VectorSubcoreMesh(core_axis_name="c", subcore_axis_name="s")
    row_ty = jax.ShapeDtypeStruct((BATCH, ROW), jnp.float32)

    HALF = ROWS_PER_WORKER // 2

    @functools.partial(
        pl.kernel,
        out_type=(row_ty, row_ty),
        mesh=mesh,
        scratch_types=[
            pltpu.VMEM((ROWS_PER_WORKER,), jnp.int32),
            pltpu.VMEM((ROWS_PER_WORKER,), jnp.int32),
            pltpu.VMEM((HALF, ROW), jnp.float32),
            pltpu.VMEM((HALF, ROW), jnp.float32),
            pltpu.SemaphoreType.DMA,
            pltpu.SemaphoreType.DMA,
        ],
    )
    def k(uidx_hbm, iidx_hbm, tab_hbm, our, oir, uix_v, iix_v, bu, bi, su, si):
        wid = lax.axis_index("s") * NUM_CORES + lax.axis_index("c")
        base = wid * ROWS_PER_WORKER
        sl = pl.ds(base, ROWS_PER_WORKER)
        pltpu.sync_copy(uidx_hbm.at[sl], uix_v)
        pltpu.sync_copy(iidx_hbm.at[sl], iix_v)
        cu = pltpu.async_copy(tab_hbm.at[uix_v.at[pl.ds(0, HALF)]], bu, su)
        ci = pltpu.async_copy(tab_hbm.at[iix_v.at[pl.ds(0, HALF)]], bi, si)
        cu.wait()
        pltpu.sync_copy(bu, our.at[pl.ds(base, HALF)])
        cu2 = pltpu.async_copy(tab_hbm.at[uix_v.at[pl.ds(HALF, HALF)]], bu, su)
        ci.wait()
        pltpu.sync_copy(bi, oir.at[pl.ds(base, HALF)])
        ci2 = pltpu.async_copy(tab_hbm.at[iix_v.at[pl.ds(HALF, HALF)]], bi, si)
        cu2.wait()
        pltpu.sync_copy(bu, our.at[pl.ds(base + HALF, HALF)])
        ci2.wait()
        pltpu.sync_copy(bi, oir.at[pl.ds(base + HALF, HALF)])

    return k(uidx, iidx, table)


def _mlp_body(ur, ir, w0, b0, w1, b1, w2, b2, w3, b3, wo, bo, out_r):
    f32 = jnp.float32
    bf16 = jnp.bfloat16
    u = ur[...]
    i = ir[...]
    um = u[:, EMBED_DIM:2 * EMBED_DIM].astype(bf16)
    im = i[:, 3 * EMBED_DIM:4 * EMBED_DIM].astype(bf16)
    w0b = w0[...].astype(bf16)
    h = (jnp.dot(um, w0b[0:EMBED_DIM, :], preferred_element_type=f32)
         + jnp.dot(im, w0b[EMBED_DIM:2 * EMBED_DIM, :], preferred_element_type=f32))
    h = jnp.maximum(h + b0[...], 0.0).astype(bf16)
    h = jnp.maximum(jnp.dot(h, w1[...].astype(bf16), preferred_element_type=f32)
                    + b1[...], 0.0).astype(bf16)
    h = jnp.maximum(jnp.dot(h, w2[...].astype(bf16), preferred_element_type=f32)
                    + b2[...], 0.0).astype(bf16)
    h = jnp.maximum(jnp.dot(h, w3[...].astype(bf16), preferred_element_type=f32)
                    + b3[...], 0.0)
    g = u[:, 0:EMBED_DIM] * i[:, 2 * EMBED_DIM:3 * EMBED_DIM]
    pred = (jnp.sum(g * wo[0:1, 0:EMBED_DIM], axis=1)
            + jnp.sum(h * wo[0:1, EMBED_DIM:EMBED_DIM + 8], axis=1)
            + bo[0, 0])
    out_r[...] = pred


def _tc_mlp(urows, irows, W0, b0, W1, b1, W2, b2, W3, b3, Wo, bo,
            block_batch=2048):
    n_blocks = BATCH // block_batch
    row_spec = pl.BlockSpec((block_batch, ROW), lambda i: (i, 0))

    def full2d(a):
        return pl.BlockSpec(a.shape, lambda i: (0, 0))

    b0r, b1r, b2r, b3r = (b.reshape(1, -1) for b in (b0, b1, b2, b3))
    bor = bo.reshape(1, 1)
    wor = Wo.reshape(1, -1)
    out = pl.pallas_call(
        _mlp_body,
        grid=(n_blocks,),
        in_specs=[row_spec, row_spec,
                  full2d(W0), full2d(b0r), full2d(W1), full2d(b1r),
                  full2d(W2), full2d(b2r), full2d(W3), full2d(b3r),
                  full2d(wor), full2d(bor)],
        out_specs=pl.BlockSpec((block_batch,), lambda i: (i,)),
        out_shape=jax.ShapeDtypeStruct((BATCH,), jnp.float32),
        compiler_params=pltpu.CompilerParams(
            dimension_semantics=("parallel",)),
    )(urows, irows,
      W0, b0r, W1, b1r, W2, b2r, W3, b3r, wor, bor)
    return out


def kernel(user_indices, item_indices, ug, ig, um, im,
           W0, b0, W1, b1, W2, b2, W3, b3, Wo, bo):
    uidx = user_indices.astype(jnp.int32)
    iidx = item_indices.astype(jnp.int32)
    table = _tc_pack(ug, um, ig, im)
    urows, irows = _sc_gather2(uidx, iidx, table)
    return _tc_mlp(urows, irows, W0, b0, W1, b1, W2, b2, W3, b3, Wo, bo)


# speedup vs baseline: 1.0067x; 1.0067x over previous
import functools

import jax
import jax.numpy as jnp
from jax import lax
from jax.experimental import pallas as pl
from jax.experimental.pallas import tpu as pltpu
from jax.experimental.pallas import tpu_sc as plsc

BATCH = 16384
EMBED_DIM = 32
ROW = 4 * EMBED_DIM
NUM_TABLE_ROWS = 100000
NUM_CORES = 2
NUM_SUBCORES = 16
NUM_WORKERS = NUM_CORES * NUM_SUBCORES
ROWS_PER_WORKER = BATCH // NUM_WORKERS


def _pack_body(ug_r, um_r, ig_r, im_r, out_r):
    dn = (((0,), (0,)), ((), ()))
    f32, bf16 = jnp.float32, jnp.bfloat16

    def tr(x, t):
        eye = jnp.eye(EMBED_DIM, ROW, EMBED_DIM * t, dtype=bf16)
        return jax.lax.dot_general(x, eye, dimension_numbers=dn,
                                   preferred_element_type=f32)

    def tr_hilo(x_r, t):
        x = x_r[...]
        hi = x.astype(bf16)
        lo = (x - hi.astype(f32)).astype(bf16)
        return tr(hi, t) + tr(lo, t)

    out_r[...] = ((tr_hilo(ug_r, 0) + tr(um_r[...].astype(bf16), 1))
                  + (tr_hilo(ig_r, 2) + tr(im_r[...].astype(bf16), 3)))


def _tc_pack(ug, um, ig, im, block_rows=8192):
    n_blocks = (NUM_TABLE_ROWS + block_rows - 1) // block_rows
    in_spec = pl.BlockSpec((EMBED_DIM, block_rows), lambda i: (0, i))
    return pl.pallas_call(
        _pack_body,
        grid=(n_blocks,),
        in_specs=[in_spec, in_spec, in_spec, in_spec],
        out_specs=pl.BlockSpec((block_rows, ROW), lambda i: (i, 0)),
        out_shape=jax.ShapeDtypeStruct((NUM_TABLE_ROWS, ROW), jnp.float32),
        compiler_params=pltpu.CompilerParams(
            dimension_semantics=("parallel",)),
    )(ug.T, um.T, ig.T, im.T)


def _sc_gather2(uidx, iidx, table):
    mesh = plsc.VectorSubcoreMesh(core_axis_name="c", subcore_axis_name="s")
    row_ty = jax.ShapeDtypeStruct((BATCH, ROW), jnp.float32)

    HALF = ROWS_PER_WORKER // 2

    @functools.partial(
        pl.kernel,
        out_type=(row_ty, row_ty),
        mesh=mesh,
        scratch_types=[
            pltpu.VMEM((ROWS_PER_WORKER,), jnp.int32),
            pltpu.VMEM((ROWS_PER_WORKER,), jnp.int32),
            pltpu.VMEM((HALF, ROW), jnp.float32),
            pltpu.VMEM((HALF, ROW), jnp.float32),
            pltpu.SemaphoreType.DMA,
            pltpu.SemaphoreType.DMA,
        ],
    )
    def k(uidx_hbm, iidx_hbm, tab_hbm, our, oir, uix_v, iix_v, bu, bi, su, si):
        wid = lax.axis_index("s") * NUM_CORES + lax.axis_index("c")
        base = wid * ROWS_PER_WORKER
        sl = pl.ds(base, ROWS_PER_WORKER)
        pltpu.sync_copy(uidx_hbm.at[sl], uix_v)
        pltpu.sync_copy(iidx_hbm.at[sl], iix_v)
        cu = pltpu.async_copy(tab_hbm.at[uix_v.at[pl.ds(0, HALF)]], bu, su)
        ci = pltpu.async_copy(tab_hbm.at[iix_v.at[pl.ds(0, HALF)]], bi, si)
        cu.wait()
        pltpu.sync_copy(bu, our.at[pl.ds(base, HALF)])
        cu2 = pltpu.async_copy(tab_hbm.at[uix_v.at[pl.ds(HALF, HALF)]], bu, su)
        ci.wait()
        pltpu.sync_copy(bi, oir.at[pl.ds(base, HALF)])
        ci2 = pltpu.async_copy(tab_hbm.at[iix_v.at[pl.ds(HALF, HALF)]], bi, si)
        cu2.wait()
        pltpu.sync_copy(bu, our.at[pl.ds(base + HALF, HALF)])
        ci2.wait()
        pltpu.sync_copy(bi, oir.at[pl.ds(base + HALF, HALF)])

    return k(uidx, iidx, table)


def _mlp_body(ur, ir, w0, b0, w1, b1, w2, b2, w3, b3, wo, bo, out_r):
    f32 = jnp.float32
    bf16 = jnp.bfloat16
    u = ur[...]
    i = ir[...]
    um = u[:, EMBED_DIM:2 * EMBED_DIM].astype(bf16)
    im = i[:, 3 * EMBED_DIM:4 * EMBED_DIM].astype(bf16)
    w0b = w0[...].astype(bf16)
    h = (jnp.dot(um, w0b[0:EMBED_DIM, :], preferred_element_type=f32)
         + jnp.dot(im, w0b[EMBED_DIM:2 * EMBED_DIM, :], preferred_element_type=f32))
    h = jnp.maximum(h + b0[...], 0.0).astype(bf16)
    h = jnp.maximum(jnp.dot(h, w1[...].astype(bf16), preferred_element_type=f32)
                    + b1[...], 0.0).astype(bf16)
    h = jnp.maximum(jnp.dot(h, w2[...].astype(bf16), preferred_element_type=f32)
                    + b2[...], 0.0).astype(bf16)
    h = jnp.maximum(jnp.dot(h, w3[...].astype(bf16), preferred_element_type=f32)
                    + b3[...], 0.0)
    g = u[:, 0:EMBED_DIM] * i[:, 2 * EMBED_DIM:3 * EMBED_DIM]
    pred = (jnp.sum(g * wo[0:1, 0:EMBED_DIM], axis=1)
            + jnp.sum(h * wo[0:1, EMBED_DIM:EMBED_DIM + 8], axis=1)
            + bo[0, 0])
    out_r[...] = pred


def _tc_mlp(urows, irows, W0, b0, W1, b1, W2, b2, W3, b3, Wo, bo,
            block_batch=2048):
    n_blocks = BATCH // block_batch
    row_spec = pl.BlockSpec((block_batch, ROW), lambda i: (i, 0))

    def full2d(a):
        return pl.BlockSpec(a.shape, lambda i: (0, 0))

    b0r, b1r, b2r, b3r = (b.reshape(1, -1) for b in (b0, b1, b2, b3))
    bor = bo.reshape(1, 1)
    wor = Wo.reshape(1, -1)
    out = pl.pallas_call(
        _mlp_body,
        grid=(n_blocks,),
        in_specs=[row_spec, row_spec,
                  full2d(W0), full2d(b0r), full2d(W1), full2d(b1r),
                  full2d(W2), full2d(b2r), full2d(W3), full2d(b3r),
                  full2d(wor), full2d(bor)],
        out_specs=pl.BlockSpec((block_batch,), lambda i: (i,)),
        out_shape=jax.ShapeDtypeStruct((BATCH,), jnp.float32),
        compiler_params=pltpu.CompilerParams(
            dimension_semantics=("parallel",)),
    )(urows, irows,
      W0, b0r, W1, b1r, W2, b2r, W3, b3r, wor, bor)
    return out


def kernel(user_indices, item_indices, ug, ig, um, im,
           W0, b0, W1, b1, W2, b2, W3, b3, Wo, bo):
    uidx = user_indices.astype(jnp.int32)
    iidx = item_indices.astype(jnp.int32)
    table = _tc_pack(ug, um, ig, im)
    urows, irows = _sc_gather2(uidx, iidx, table)
    return _tc_mlp(urows, irows, W0, b0, W1, b1, W2, b2, W3, b3, Wo, bo)
